# Initial kernel scaffold; baseline (speedup 1.0000x reference)
import jax
import jax.numpy as jnp
from jax.experimental import pallas as pl


def kernel(latents, embedding_weight):
    raise NotImplementedError("write your pallas kernel here")


# speedup vs baseline: 4.1933x; 4.1933x over previous
import functools

import jax
import jax.numpy as jnp
from jax.experimental import pallas as pl
from jax.experimental.pallas import tpu as pltpu

K = 64
D = 64
BETA = 0.25
BLOCK = 8192


def _vq_block_kernel(x_ref, e_ref, oh_ref, q_ref, idx_ref, loss_ref):
    i = pl.program_id(0)
    x = x_ref[...]
    e = e_ref[...]

    w2 = jnp.sum(e * e, axis=0, keepdims=True)
    x2 = jnp.sum(x * x, axis=1, keepdims=True)
    s = jax.lax.dot_general(x, e, (((1,), (1,)), ((), ())),
                            preferred_element_type=jnp.float32)
    scores = (x2 + w2) - 2.0 * s

    m = jnp.min(scores, axis=1, keepdims=True)
    eq = (scores == m).astype(jnp.float32)

    kk = jax.lax.broadcasted_iota(jnp.int32, (K, K), 0)
    ll = jax.lax.broadcasted_iota(jnp.int32, (K, K), 1)
    lower = (kk < ll).astype(jnp.float32)
    prior = jax.lax.dot_general(eq, lower, (((1,), (0,)), ((), ())),
                                preferred_element_type=jnp.float32)
    one_hot = eq * (prior == 0.0).astype(jnp.float32)

    col = jax.lax.broadcasted_iota(jnp.int32, (K, 1), 0).astype(jnp.float32)
    idxf = jax.lax.dot_general(one_hot, col, (((1,), (0,)), ((), ())),
                               preferred_element_type=jnp.float32)
    idx = idxf.astype(jnp.int32)

    q = jax.lax.dot_general(one_hot, e, (((1,), (0,)), ((), ())),
                            preferred_element_type=jnp.float32)

    oh_ref[...] = one_hot
    q_ref[...] = q
    idx_ref[...] = idx

    diff = q - x
    d2 = diff * diff
    ones_col = jnp.ones((D, 1), jnp.float32)
    lane_sums = jax.lax.dot_general(d2, ones_col, (((1,), (0,)), ((), ())),
                                    preferred_element_type=jnp.float32)
    ones_row = jnp.ones((1, x.shape[0]), jnp.float32)
    partial = jax.lax.dot_general(ones_row, lane_sums, (((1,), (0,)), ((), ())),
                                  preferred_element_type=jnp.float32)

    @pl.when(i == 0)
    def _():
        loss_ref[...] = jnp.zeros_like(loss_ref)

    loss_ref[...] += partial


def kernel(latents, embedding_weight):
    shape = latents.shape
    flat = latents.reshape(-1, D)
    n = flat.shape[0]
    nb = n // BLOCK

    one_hot, quant, idx, loss = pl.pallas_call(
        _vq_block_kernel,
        grid=(nb,),
        in_specs=[
            pl.BlockSpec((BLOCK, D), lambda i: (i, 0)),
            pl.BlockSpec((K, D), lambda i: (0, 0)),
        ],
        out_specs=[
            pl.BlockSpec((BLOCK, K), lambda i: (i, 0)),
            pl.BlockSpec((BLOCK, D), lambda i: (i, 0)),
            pl.BlockSpec((BLOCK, 1), lambda i: (i, 0)),
            pl.BlockSpec((1, 1), lambda i: (0, 0)),
        ],
        out_shape=[
            jax.ShapeDtypeStruct((n, K), jnp.float32),
            jax.ShapeDtypeStruct((n, D), jnp.float32),
            jax.ShapeDtypeStruct((n, 1), jnp.int32),
            jax.ShapeDtypeStruct((1, 1), jnp.float32),
        ],
        compiler_params=pltpu.CompilerParams(
            dimension_semantics=("arbitrary",),
        ),
    )(flat, embedding_weight)

    quantized = quant.reshape(shape)
    indices = idx.reshape(shape[0], shape[1], shape[2])[:, None, :, :]
    vq_loss = loss[0, 0] * ((1.0 + BETA) / (n * D))
    return (quantized, vq_loss, one_hot, indices)
